# baseline (speedup 1.0000x reference)
import functools

import jax
import jax.numpy as jnp
from jax import lax
from jax.experimental import pallas as pl
from jax.experimental.pallas import tpu as pltpu


def _build_maps_kernel(offt_ref, offn_ref, ctw_ref, cnwt_ref, a_ref,
                       *, n_dw, n_pred, n_drift):
    L = n_pred + 2 * n_drift
    f32 = jnp.float32
    i32 = jnp.int32
    j_io = lax.broadcasted_iota(i32, (1, n_pred), 1).astype(f32)
    k_io = lax.broadcasted_iota(i32, (L, n_pred), 0).astype(f32)
    r_io = lax.broadcasted_iota(i32, (n_pred, n_pred), 0).astype(f32)
    c_io = lax.broadcasted_iota(i32, (n_pred, n_pred), 1).astype(f32)
    m_right = (c_io - r_io == 1.0).astype(f32)
    m_left = (r_io - c_io == 1.0).astype(f32)

    for i in range(n_dw):
        pos_t = jnp.tanh(offt_ref[pl.ds(i, 1), :]) * float(n_drift) \
            + j_io + float(n_drift)
        key_t = jnp.floor(pos_t)
        fr_t = pos_t - key_t
        s_t = (k_io == key_t) * (1.0 - fr_t) + (k_io == key_t + 1.0) * fr_t
        ct0 = ctw_ref[pl.ds(i, 1), pl.ds(0, 1)]
        ct1 = ctw_ref[pl.ds(i, 1), pl.ds(1, 1)]
        ct2 = ctw_ref[pl.ds(i, 1), pl.ds(2, 1)]
        a_t = s_t * ct1 \
            + jnp.dot(s_t * ct0, m_right, preferred_element_type=f32,
                      precision=lax.Precision.HIGHEST) \
            + jnp.dot(s_t * ct2, m_left, preferred_element_type=f32,
                      precision=lax.Precision.HIGHEST)
        a_ref[pl.ds(i * L, L), pl.ds(0, n_pred)] = a_t

        pos_n = jnp.tanh(offn_ref[pl.ds(i, 1), :]) * float(n_drift) \
            + j_io + float(n_drift)
        key_n = jnp.floor(pos_n)
        fr_n = pos_n - key_n
        s_n = (k_io == key_n) * (1.0 - fr_n) + (k_io == key_n + 1.0) * fr_n
        cn_i = cnwt_ref[pl.ds(i * n_pred, n_pred), :]
        a_ref[pl.ds(i * L, L), pl.ds(n_pred, n_pred)] = jnp.dot(
            s_n, cn_i, preferred_element_type=f32,
            precision=lax.Precision.HIGHEST)


def _stream_kernel(x0_ref, x1_ref, ctrl_ref, bp_ref, a_ref, biases_ref,
                   w_ref, out_ref, *, n_pred, hb):
    f32 = jnp.float32
    bias_t = biases_ref[pl.ds(0, 1), :]
    bias_n = biases_ref[pl.ds(1, 1), :]
    act = jnp.dot(ctrl_ref[0], w_ref[:], preferred_element_type=f32) \
        + bp_ref[:]
    g = jax.nn.sigmoid(act)
    x_halves = (x0_ref[0], lax.bitcast_convert_type(x1_ref[0], f32))
    for h, x in enumerate(x_halves):
        p = jnp.dot(x, a_ref[:], preferred_element_type=f32)
        pt = p[:, 0:n_pred] + bias_t
        pn = p[:, n_pred:2 * n_pred] + bias_n
        out_ref[0, pl.ds(h * hb, hb), :] = \
            pt + g[h * hb:(h + 1) * hb, :] * (pn - pt)


def kernel(inp, ctrl, offset_t, offset_n, conv_t_w, conv_t_b, conv_n_w,
           conv_n_b, W, b_param):
    n_dw, n_pred = offset_t.shape
    b, n = inp.shape[0], inp.shape[1]
    L = inp.shape[2] // n_dw
    n_drift = (L - n_pred) // 2

    cnwt = jnp.transpose(conv_n_w, (2, 1, 0)).reshape(n_dw * n_pred, n_pred)
    ctw = conv_t_w.reshape(n_dw, 3)
    biases = jnp.stack([
        jnp.broadcast_to(conv_t_b, (n_pred,)),
        conv_n_b,
    ])

    a = pl.pallas_call(
        functools.partial(_build_maps_kernel, n_dw=n_dw, n_pred=n_pred,
                          n_drift=n_drift),
        out_shape=jax.ShapeDtypeStruct((n_dw * L, 2 * n_pred), jnp.float32),
    )(offset_t, offset_n, ctw, cnwt)

    hb = n // 2
    grid = (b,)
    inp_u = lax.bitcast_convert_type(inp, jnp.uint32)

    out = pl.pallas_call(
        functools.partial(_stream_kernel, n_pred=n_pred, hb=hb),
        grid=grid,
        in_specs=[
            pl.BlockSpec((1, hb, n_dw * L), lambda i: (i, 0, 0)),
            pl.BlockSpec((1, hb, n_dw * L), lambda i: (i, 1, 0)),
            pl.BlockSpec((1, n, n_pred), lambda i: (i, 0, 0)),
            pl.BlockSpec((n, n_pred), lambda i: (0, 0)),
            pl.BlockSpec((n_dw * L, 2 * n_pred), lambda i: (0, 0)),
            pl.BlockSpec((2, n_pred), lambda i: (0, 0)),
            pl.BlockSpec((n_pred, n_pred), lambda i: (0, 0)),
        ],
        out_specs=pl.BlockSpec((1, n, n_pred), lambda i: (i, 0, 0)),
        out_shape=jax.ShapeDtypeStruct((b, n, n_pred), jnp.float32),
        compiler_params=pltpu.CompilerParams(
            dimension_semantics=("arbitrary",)),
    )(inp, inp_u, ctrl, b_param, a, biases, W)

    return out


# speedup vs baseline: 1.5560x; 1.5560x over previous
import functools

import jax
import jax.numpy as jnp
from jax import lax
from jax.experimental import pallas as pl
from jax.experimental.pallas import tpu as pltpu


def _build_maps_kernel(offt_ref, offn_ref, ctw_ref, cnwt_ref, a_ref,
                       *, n_dw, n_pred, n_drift):
    L = n_pred + 2 * n_drift
    f32 = jnp.float32
    i32 = jnp.int32
    j_io = lax.broadcasted_iota(i32, (1, n_pred), 1).astype(f32)
    k_io = lax.broadcasted_iota(i32, (L, n_pred), 0).astype(f32)
    r_io = lax.broadcasted_iota(i32, (n_pred, n_pred), 0).astype(f32)
    c_io = lax.broadcasted_iota(i32, (n_pred, n_pred), 1).astype(f32)
    m_right = (c_io - r_io == 1.0).astype(f32)
    m_left = (r_io - c_io == 1.0).astype(f32)

    for i in range(n_dw):
        pos_t = jnp.tanh(offt_ref[pl.ds(i, 1), :]) * float(n_drift) \
            + j_io + float(n_drift)
        key_t = jnp.floor(pos_t)
        fr_t = pos_t - key_t
        s_t = (k_io == key_t) * (1.0 - fr_t) + (k_io == key_t + 1.0) * fr_t
        ct0 = ctw_ref[pl.ds(i, 1), pl.ds(0, 1)]
        ct1 = ctw_ref[pl.ds(i, 1), pl.ds(1, 1)]
        ct2 = ctw_ref[pl.ds(i, 1), pl.ds(2, 1)]
        a_t = s_t * ct1 \
            + jnp.dot(s_t * ct0, m_right, preferred_element_type=f32,
                      precision=lax.Precision.HIGHEST) \
            + jnp.dot(s_t * ct2, m_left, preferred_element_type=f32,
                      precision=lax.Precision.HIGHEST)
        a_ref[pl.ds(i * L, L), pl.ds(0, n_pred)] = a_t

        pos_n = jnp.tanh(offn_ref[pl.ds(i, 1), :]) * float(n_drift) \
            + j_io + float(n_drift)
        key_n = jnp.floor(pos_n)
        fr_n = pos_n - key_n
        s_n = (k_io == key_n) * (1.0 - fr_n) + (k_io == key_n + 1.0) * fr_n
        cn_i = cnwt_ref[pl.ds(i * n_pred, n_pred), :]
        a_ref[pl.ds(i * L, L), pl.ds(n_pred, n_pred)] = jnp.dot(
            s_n, cn_i, preferred_element_type=f32,
            precision=lax.Precision.HIGHEST)


NBUF = 4


def _stream_kernel(x_hbm, ctrl_ref, bp_ref, a_ref, biases_ref, w_ref,
                   out_ref, xbufs, sems, *, n_pred, cr, ch):
    f32 = jnp.float32
    i = pl.program_id(0)
    nsteps = pl.num_programs(0)

    def start_copy(step):
        slot = jax.lax.rem(step, NBUF)
        pltpu.make_async_copy(
            x_hbm.at[step // ch, pl.ds(jax.lax.rem(step, ch) * cr, cr), :],
            xbufs.at[slot], sems.at[slot]).start()

    @pl.when(i == 0)
    def _prologue():
        for k in range(NBUF):
            start_copy(jnp.int32(k))

    @pl.when((i > 0) & (i + NBUF - 1 < nsteps))
    def _refill():
        start_copy(i + NBUF - 1)

    slot = jax.lax.rem(i, NBUF)
    pltpu.make_async_copy(
        x_hbm.at[i // ch, pl.ds(jax.lax.rem(i, ch) * cr, cr), :],
        xbufs.at[slot], sems.at[slot]).wait()

    x = xbufs[slot]
    p = jnp.dot(x, a_ref[:], preferred_element_type=f32)
    bias_t = biases_ref[pl.ds(0, 1), :]
    bias_n = biases_ref[pl.ds(1, 1), :]
    pt = p[:, 0:n_pred] + bias_t
    pn = p[:, n_pred:2 * n_pred] + bias_n
    act = jnp.dot(ctrl_ref[0], w_ref[:], preferred_element_type=f32) \
        + bp_ref[:]
    g = jax.nn.sigmoid(act)
    out_ref[0] = pt + g * (pn - pt)


def make_stream_call(b, n, n_dw, L, n_pred):
    ch = 2 if n % 16 == 0 else 1
    cr = n // ch
    return pl.pallas_call(
        functools.partial(_stream_kernel, n_pred=n_pred, cr=cr, ch=ch),
        grid=(b * ch,),
        in_specs=[
            pl.BlockSpec(memory_space=pltpu.MemorySpace.HBM),
            pl.BlockSpec((1, cr, n_pred), lambda i: (i // ch, i % ch, 0)),
            pl.BlockSpec((cr, n_pred), lambda i: (i % ch, 0)),
            pl.BlockSpec((n_dw * L, 2 * n_pred), lambda i: (0, 0)),
            pl.BlockSpec((2, n_pred), lambda i: (0, 0)),
            pl.BlockSpec((n_pred, n_pred), lambda i: (0, 0)),
        ],
        out_specs=pl.BlockSpec((1, cr, n_pred), lambda i: (i // ch, i % ch, 0)),
        out_shape=jax.ShapeDtypeStruct((b, n, n_pred), jnp.float32),
        scratch_shapes=[
            pltpu.VMEM((NBUF, cr, n_dw * L), jnp.float32),
            pltpu.SemaphoreType.DMA((NBUF,)),
        ],
        compiler_params=pltpu.CompilerParams(
            dimension_semantics=("arbitrary",)),
    )


def kernel(inp, ctrl, offset_t, offset_n, conv_t_w, conv_t_b, conv_n_w,
           conv_n_b, W, b_param):
    n_dw, n_pred = offset_t.shape
    b, n = inp.shape[0], inp.shape[1]
    L = inp.shape[2] // n_dw
    n_drift = (L - n_pred) // 2

    cnwt = jnp.transpose(conv_n_w, (2, 1, 0)).reshape(n_dw * n_pred, n_pred)
    ctw = conv_t_w.reshape(n_dw, 3)
    biases = jnp.stack([
        jnp.broadcast_to(conv_t_b, (n_pred,)),
        conv_n_b,
    ])

    a = pl.pallas_call(
        functools.partial(_build_maps_kernel, n_dw=n_dw, n_pred=n_pred,
                          n_drift=n_drift),
        out_shape=jax.ShapeDtypeStruct((n_dw * L, 2 * n_pred), jnp.float32),
    )(offset_t, offset_n, ctw, cnwt)

    out = make_stream_call(b, n, n_dw, L, n_pred)(
        inp, ctrl, b_param, a, biases, W)
    return out


# speedup vs baseline: 1.6440x; 1.0566x over previous
import functools

import jax
import jax.numpy as jnp
from jax import lax
from jax.experimental import pallas as pl
from jax.experimental.pallas import tpu as pltpu


def _build_maps_kernel(offt_ref, offn_ref, ctw_ref, cnwt_ref, a_ref,
                       *, n_dw, n_pred, n_drift):
    L = n_pred + 2 * n_drift
    f32 = jnp.float32
    i32 = jnp.int32
    j_io = lax.broadcasted_iota(i32, (1, n_pred), 1).astype(f32)
    k_io = lax.broadcasted_iota(i32, (L, n_pred), 0).astype(f32)
    r_io = lax.broadcasted_iota(i32, (n_pred, n_pred), 0).astype(f32)
    c_io = lax.broadcasted_iota(i32, (n_pred, n_pred), 1).astype(f32)
    m_right = (c_io - r_io == 1.0).astype(f32)
    m_left = (r_io - c_io == 1.0).astype(f32)

    for i in range(n_dw):
        pos_t = jnp.tanh(offt_ref[pl.ds(i, 1), :]) * float(n_drift) \
            + j_io + float(n_drift)
        key_t = jnp.floor(pos_t)
        fr_t = pos_t - key_t
        s_t = (k_io == key_t) * (1.0 - fr_t) + (k_io == key_t + 1.0) * fr_t
        ct0 = ctw_ref[pl.ds(i, 1), pl.ds(0, 1)]
        ct1 = ctw_ref[pl.ds(i, 1), pl.ds(1, 1)]
        ct2 = ctw_ref[pl.ds(i, 1), pl.ds(2, 1)]
        a_t = s_t * ct1 \
            + jnp.dot(s_t * ct0, m_right, preferred_element_type=f32,
                      precision=lax.Precision.HIGHEST) \
            + jnp.dot(s_t * ct2, m_left, preferred_element_type=f32,
                      precision=lax.Precision.HIGHEST)
        a_ref[pl.ds(i * L, L), pl.ds(0, n_pred)] = a_t

        pos_n = jnp.tanh(offn_ref[pl.ds(i, 1), :]) * float(n_drift) \
            + j_io + float(n_drift)
        key_n = jnp.floor(pos_n)
        fr_n = pos_n - key_n
        s_n = (k_io == key_n) * (1.0 - fr_n) + (k_io == key_n + 1.0) * fr_n
        cn_i = cnwt_ref[pl.ds(i * n_pred, n_pred), :]
        a_ref[pl.ds(i * L, L), pl.ds(n_pred, n_pred)] = jnp.dot(
            s_n, cn_i, preferred_element_type=f32,
            precision=lax.Precision.HIGHEST)


def _stream_kernel(x_ref, ctrl_ref, bp_ref, a_ref, biases_ref, w_ref,
                   out_ref, *, n_pred, bb):
    f32 = jnp.float32
    bias_t = biases_ref[pl.ds(0, 1), :]
    bias_n = biases_ref[pl.ds(1, 1), :]
    for ib in range(bb):
        p = jnp.dot(x_ref[ib], a_ref[:], preferred_element_type=f32)
        pt = p[:, 0:n_pred] + bias_t
        pn = p[:, n_pred:2 * n_pred] + bias_n
        act = jnp.dot(ctrl_ref[ib], w_ref[:], preferred_element_type=f32) \
            + bp_ref[:]
        g = jax.nn.sigmoid(act)
        out_ref[ib] = pt + g * (pn - pt)


def kernel(inp, ctrl, offset_t, offset_n, conv_t_w, conv_t_b, conv_n_w,
           conv_n_b, W, b_param):
    n_dw, n_pred = offset_t.shape
    b, n = inp.shape[0], inp.shape[1]
    L = inp.shape[2] // n_dw
    n_drift = (L - n_pred) // 2

    cnwt = jnp.transpose(conv_n_w, (2, 1, 0)).reshape(n_dw * n_pred, n_pred)
    ctw = conv_t_w.reshape(n_dw, 3)
    biases = jnp.stack([
        jnp.broadcast_to(conv_t_b, (n_pred,)),
        conv_n_b,
    ])

    a = pl.pallas_call(
        functools.partial(_build_maps_kernel, n_dw=n_dw, n_pred=n_pred,
                          n_drift=n_drift),
        out_shape=jax.ShapeDtypeStruct((n_dw * L, 2 * n_pred), jnp.float32),
    )(offset_t, offset_n, ctw, cnwt)

    bb = 1
    grid = (b // bb,)

    out = pl.pallas_call(
        functools.partial(_stream_kernel, n_pred=n_pred, bb=bb),
        grid=grid,
        in_specs=[
            pl.BlockSpec((bb, n, n_dw * L), lambda i: (i, 0, 0)),
            pl.BlockSpec((bb, n, n_pred), lambda i: (i, 0, 0)),
            pl.BlockSpec((n, n_pred), lambda i: (0, 0)),
            pl.BlockSpec((n_dw * L, 2 * n_pred), lambda i: (0, 0)),
            pl.BlockSpec((2, n_pred), lambda i: (0, 0)),
            pl.BlockSpec((n_pred, n_pred), lambda i: (0, 0)),
        ],
        out_specs=pl.BlockSpec((bb, n, n_pred), lambda i: (i, 0, 0)),
        out_shape=jax.ShapeDtypeStruct((b, n, n_pred), jnp.float32),
        compiler_params=pltpu.CompilerParams(
            dimension_semantics=("arbitrary",)),
    )(inp, ctrl, b_param, a, biases, W)

    return out


# speedup vs baseline: 1.6571x; 1.0080x over previous
import functools

import jax
import jax.numpy as jnp
from jax import lax
from jax.experimental import pallas as pl
from jax.experimental.pallas import tpu as pltpu


def _fused_kernel(x_ref, ctrl_ref, bp_ref, offt_ref, offn_ref, ctw_ref,
                  cnwt_ref, biases_ref, w_ref, out_ref, a_s,
                  *, n_dw, n_pred, n_drift):
    L = n_pred + 2 * n_drift
    f32 = jnp.float32

    @pl.when(pl.program_id(0) == 0)
    def _build_maps():
        i32 = jnp.int32
        j_io = lax.broadcasted_iota(i32, (1, n_pred), 1).astype(f32)
        k_io = lax.broadcasted_iota(i32, (L, n_pred), 0).astype(f32)
        r_io = lax.broadcasted_iota(i32, (n_pred, n_pred), 0).astype(f32)
        c_io = lax.broadcasted_iota(i32, (n_pred, n_pred), 1).astype(f32)
        m_right = (c_io - r_io == 1.0).astype(f32)
        m_left = (r_io - c_io == 1.0).astype(f32)

        for i in range(n_dw):
            pos_t = jnp.tanh(offt_ref[pl.ds(i, 1), :]) * float(n_drift) \
                + j_io + float(n_drift)
            key_t = jnp.floor(pos_t)
            fr_t = pos_t - key_t
            s_t = (k_io == key_t) * (1.0 - fr_t) + (k_io == key_t + 1.0) * fr_t
            ct0 = ctw_ref[pl.ds(i, 1), pl.ds(0, 1)]
            ct1 = ctw_ref[pl.ds(i, 1), pl.ds(1, 1)]
            ct2 = ctw_ref[pl.ds(i, 1), pl.ds(2, 1)]
            a_t = s_t * ct1 \
                + jnp.dot(s_t * ct0, m_right, preferred_element_type=f32,
                          precision=lax.Precision.HIGHEST) \
                + jnp.dot(s_t * ct2, m_left, preferred_element_type=f32,
                          precision=lax.Precision.HIGHEST)
            a_s[pl.ds(i * L, L), pl.ds(0, n_pred)] = a_t

            pos_n = jnp.tanh(offn_ref[pl.ds(i, 1), :]) * float(n_drift) \
                + j_io + float(n_drift)
            key_n = jnp.floor(pos_n)
            fr_n = pos_n - key_n
            s_n = (k_io == key_n) * (1.0 - fr_n) + (k_io == key_n + 1.0) * fr_n
            cn_i = cnwt_ref[pl.ds(i * n_pred, n_pred), :]
            a_s[pl.ds(i * L, L), pl.ds(n_pred, n_pred)] = jnp.dot(
                s_n, cn_i, preferred_element_type=f32,
                precision=lax.Precision.HIGHEST)

    p = jnp.dot(x_ref[0], a_s[:], preferred_element_type=f32)
    bias_t = biases_ref[pl.ds(0, 1), :]
    bias_n = biases_ref[pl.ds(1, 1), :]
    pt = p[:, 0:n_pred] + bias_t
    pn = p[:, n_pred:2 * n_pred] + bias_n
    act = jnp.dot(ctrl_ref[0], w_ref[:], preferred_element_type=f32) \
        + bp_ref[:]
    g = jax.nn.sigmoid(act)
    out_ref[0] = pt + g * (pn - pt)


def kernel(inp, ctrl, offset_t, offset_n, conv_t_w, conv_t_b, conv_n_w,
           conv_n_b, W, b_param):
    n_dw, n_pred = offset_t.shape
    b, n = inp.shape[0], inp.shape[1]
    L = inp.shape[2] // n_dw
    n_drift = (L - n_pred) // 2

    cnwt = jnp.transpose(conv_n_w, (2, 1, 0)).reshape(n_dw * n_pred, n_pred)
    ctw = conv_t_w.reshape(n_dw, 3)
    biases = jnp.stack([
        jnp.broadcast_to(conv_t_b, (n_pred,)),
        conv_n_b,
    ])

    out = pl.pallas_call(
        functools.partial(_fused_kernel, n_dw=n_dw, n_pred=n_pred,
                          n_drift=n_drift),
        grid=(b,),
        in_specs=[
            pl.BlockSpec((1, n, n_dw * L), lambda i: (i, 0, 0)),
            pl.BlockSpec((1, n, n_pred), lambda i: (i, 0, 0)),
            pl.BlockSpec((n, n_pred), lambda i: (0, 0)),
            pl.BlockSpec((n_dw, n_pred), lambda i: (0, 0)),
            pl.BlockSpec((n_dw, n_pred), lambda i: (0, 0)),
            pl.BlockSpec((n_dw, 3), lambda i: (0, 0)),
            pl.BlockSpec((n_dw * n_pred, n_pred), lambda i: (0, 0)),
            pl.BlockSpec((2, n_pred), lambda i: (0, 0)),
            pl.BlockSpec((n_pred, n_pred), lambda i: (0, 0)),
        ],
        out_specs=pl.BlockSpec((1, n, n_pred), lambda i: (i, 0, 0)),
        out_shape=jax.ShapeDtypeStruct((b, n, n_pred), jnp.float32),
        scratch_shapes=[
            pltpu.VMEM((n_dw * L, 2 * n_pred), jnp.float32),
        ],
        compiler_params=pltpu.CompilerParams(
            dimension_semantics=("arbitrary",)),
    )(inp, ctrl, b_param, offset_t, offset_n, ctw, cnwt, biases, W)

    return out
